# baseline (speedup 1.0000x reference)
import jax
import jax.numpy as jnp
from jax import lax
from jax.experimental import pallas as pl
from jax.experimental.pallas import tpu as pltpu
from jax.experimental.pallas import tpu_sc as plsc

NC = 2
NS = 16
L = 16
NBINS = 64


def _pad16(n):
    return (n + 15) // 16 * 16


def _make_body(B, NWORDS, VOCAB, C, C2, KEEP_LT):
    GROUPS = NWORDS // L
    UNROLL = 1
    DOCS_PER_WORKER = B // (NC * NS)
    KEPT_PAD = NWORDS + 4 * L

    def body(doc_hbm, wht_hbm, wfi_hbm, bht_hbm, bfi_hbm, out_hbm,
             doc_v, wht_v, wfi_v, bht_v, bfi_v, hist_v, kept_v, cnt_v):
        wid = lax.axis_index("s") * NC + lax.axis_index("c")

        pltpu.sync_copy(wht_hbm, wht_v)
        pltpu.sync_copy(wfi_hbm, wfi_v)
        pltpu.sync_copy(bht_hbm, bht_v)
        pltpu.sync_copy(bfi_hbm, bfi_v)

        iot = lax.broadcasted_iota(jnp.int32, (L,), 0)
        laneH = iot * NBINS
        lane4 = iot * 4
        ones = jnp.ones((L,), jnp.int32)
        zeros = jnp.zeros((L,), jnp.int32)

        bhv = [bht_v[pl.ds(16 * i, L)] for i in range(bht_v.shape[0] // L)]
        bfv = [bfi_v[pl.ds(16 * i, L)] for i in range(bfi_v.shape[0] // L)]

        def _elem(vs, pos):
            return vs[pos // L][pos % L]

        bgA = [jnp.full((L,), _elem(bhv, k * (2 * C2) + s * 2 + 0), jnp.int32)
               for k in range(2) for s in range(C2)]
        bgB = [jnp.full((L,), _elem(bhv, k * (2 * C2) + s * 2 + 1), jnp.int32)
               for k in range(2) for s in range(C2)]
        bgF = [jnp.full((L,), _elem(bfv, k * C2 + s), jnp.int32)
               for k in range(2) for s in range(C2)]

        c1 = jnp.uint32(3432918353)
        c2 = jnp.uint32(461845907)

        for k in range(DOCS_PER_WORKER):
            d = wid * DOCS_PER_WORKER + k
            pltpu.sync_copy(doc_hbm.at[pl.ds(d * (NWORDS * 4), NWORDS * 4)],
                            doc_v)
            for r in range(16 * NBINS // L):
                hist_v[pl.ds(r * L, L)] = zeros

            def group(g, off):
                base = g * (L * 4) + lane4
                p = [plsc.load_gather(doc_v, [base + j]) for j in range(4)]
                h = jnp.zeros((L,), jnp.uint32)
                for j in range(4):
                    h = h ^ (p[j].astype(jnp.uint32) * c1 * c2)
                idxw = (h % jnp.uint32(VOCAB)).astype(jnp.int32)
                iK = idxw * (4 * C)
                iF = idxw * C
                ind = zeros
                for s in range(C):
                    t0 = plsc.load_gather(wht_v, [iK + (4 * s + 0)])
                    t1 = plsc.load_gather(wht_v, [iK + (4 * s + 1)])
                    t2 = plsc.load_gather(wht_v, [iK + (4 * s + 2)])
                    t3 = plsc.load_gather(wht_v, [iK + (4 * s + 3)])
                    fv = plsc.load_gather(wfi_v, [iF + s])
                    m = (p[0] == t0) & (p[1] == t1) & (p[2] == t2) & (p[3] == t3)
                    ind = ind + jnp.where(m, fv, 0)
                plsc.addupdate_scatter(hist_v, [laneH + ind], ones)
                keep = ind < KEEP_LT
                plsc.store_compressed(kept_v.at[pl.ds(off, L)], ind, mask=keep)
                nkeep = jnp.sum(keep, dtype=jnp.int32).astype(jnp.int32)
                return (off + nkeep).astype(jnp.int32)

            def groupN(gq, off):
                g = gq * UNROLL
                for u in range(UNROLL):
                    off = group(g + u, off)
                return off

            off = lax.fori_loop(jnp.int32(0), jnp.int32(GROUPS // UNROLL),
                                groupN, jnp.int32(0))

            npairs = jnp.maximum(off - 1, 0)
            ngrp2 = ((npairs + (2 * L - 1)) // (2 * L)).astype(jnp.int32)

            def pone(base):
                ia = iot + base
                a = plsc.load_gather(kept_v, [ia])
                b = plsc.load_gather(kept_v, [ia + 1])
                valid = ia < npairs
                bind = zeros
                for s in range(2 * C2):
                    bind = bind + jnp.where((a == bgA[s]) & (b == bgB[s]), bgF[s], 0)
                plsc.addupdate_scatter(hist_v, [laneH + bind], ones, mask=valid)

            def pgroup(i, carry):
                pone(i * (2 * L))
                pone(i * (2 * L) + L)
                return carry

            lax.fori_loop(jnp.int32(0), ngrp2, pgroup, jnp.int32(0))

            acc = [zeros] * (NBINS // L)
            for r in range(16):
                for c in range(NBINS // L):
                    acc[c] = acc[c] + hist_v[pl.ds(r * NBINS + c * L, L)]
            for c in range(NBINS // L):
                cnt_v[pl.ds(c * L, L)] = acc[c]
            pltpu.sync_copy(cnt_v, out_hbm.at[d])

    return body, KEPT_PAD


def kernel(documents, word_hash_table, word_feature_indices,
           bigram_hash_table, bigram_feature_indices):
    B, NWORDS, MWL = documents.shape
    VOCAB, C, _ = word_hash_table.shape
    _, C2, _ = bigram_hash_table.shape
    NF = 32
    KEEP_LT = NF + 1

    docs = lax.bitcast_convert_type(
        documents.astype(jnp.uint8).reshape(B, NWORDS, MWL // 4, 4),
        jnp.int32).reshape(-1)
    whtf = word_hash_table.astype(jnp.int32).reshape(-1)
    wfif = word_feature_indices.astype(jnp.int32).reshape(-1)
    bhtf = bigram_hash_table.astype(jnp.int32).reshape(-1)
    bfif = bigram_feature_indices.astype(jnp.int32).reshape(-1)
    wht = jnp.zeros((_pad16(whtf.shape[0]),), jnp.int32).at[:whtf.shape[0]].set(whtf)
    wfi = jnp.zeros((_pad16(wfif.shape[0]),), jnp.int32).at[:wfif.shape[0]].set(wfif)
    bht = jnp.zeros((_pad16(bhtf.shape[0]),), jnp.int32).at[:bhtf.shape[0]].set(bhtf)
    bfi = jnp.zeros((_pad16(bfif.shape[0]),), jnp.int32).at[:bfif.shape[0]].set(bfif)

    body, KEPT_PAD = _make_body(B, NWORDS, VOCAB, C, C2, KEEP_LT)
    mesh = plsc.VectorSubcoreMesh(core_axis_name="c", subcore_axis_name="s")
    run = pl.kernel(
        body,
        mesh=mesh,
        compiler_params=pltpu.CompilerParams(use_tc_tiling_on_sc=False,
                                             needs_layout_passes=False),
        out_type=jax.ShapeDtypeStruct((B, NBINS), jnp.int32),
        scratch_types=[
            pltpu.VMEM((NWORDS * 4,), jnp.int32),
            pltpu.VMEM((wht.shape[0],), jnp.int32),
            pltpu.VMEM((wfi.shape[0],), jnp.int32),
            pltpu.VMEM((bht.shape[0],), jnp.int32),
            pltpu.VMEM((bfi.shape[0],), jnp.int32),
            pltpu.VMEM((16 * NBINS,), jnp.int32),
            pltpu.VMEM((KEPT_PAD,), jnp.int32),
            pltpu.VMEM((NBINS,), jnp.int32),
        ],
    )
    padded = run(docs, wht, wfi, bht, bfi)
    return padded[:, 1:1 + NF].astype(jnp.int64)


# speedup vs baseline: 3.2906x; 3.2906x over previous
import jax
import jax.numpy as jnp
from jax import lax
from jax.experimental import pallas as pl
from jax.experimental.pallas import tpu as pltpu
from jax.experimental.pallas import tpu_sc as plsc

NC = 2
NS = 16
L = 16
NBINS = 64


def _pad16(n):
    return (n + 15) // 16 * 16


def _make_body(B, NWORDS, VOCAB, C, C2, KEEP_LT):
    GROUPS = NWORDS // L
    UNROLL = 1
    DOCS_PER_WORKER = B // (NC * NS)
    KEPT_PAD = NWORDS + 4 * L

    def body(doc_hbm, wht_hbm, wfi_hbm, bht_hbm, bfi_hbm, out_hbm,
             doc_v, wht_v, wfi_v, bht_v, bfi_v, hist_v, kept_v, cnt_v):
        wid = lax.axis_index("s") * NC + lax.axis_index("c")

        pltpu.sync_copy(wht_hbm, wht_v)
        pltpu.sync_copy(wfi_hbm, wfi_v)
        pltpu.sync_copy(bht_hbm, bht_v)
        pltpu.sync_copy(bfi_hbm, bfi_v)

        iot = lax.broadcasted_iota(jnp.int32, (L,), 0)
        laneH = iot * NBINS
        lane4 = iot * 4
        ones = jnp.ones((L,), jnp.int32)
        zeros = jnp.zeros((L,), jnp.int32)

        bhv = [bht_v[pl.ds(16 * i, L)] for i in range(bht_v.shape[0] // L)]
        bfv = [bfi_v[pl.ds(16 * i, L)] for i in range(bfi_v.shape[0] // L)]

        def _elem(vs, pos):
            return vs[pos // L][pos % L]

        bgA = [jnp.full((L,), _elem(bhv, k * (2 * C2) + s * 2 + 0), jnp.int32)
               for k in range(2) for s in range(C2)]
        bgB = [jnp.full((L,), _elem(bhv, k * (2 * C2) + s * 2 + 1), jnp.int32)
               for k in range(2) for s in range(C2)]
        bgF = [jnp.full((L,), _elem(bfv, k * C2 + s), jnp.int32)
               for k in range(2) for s in range(C2)]

        c1 = jnp.uint32(3432918353)
        c2 = jnp.uint32(461845907)

        for k in range(DOCS_PER_WORKER):
            d = wid * DOCS_PER_WORKER + k
            pltpu.sync_copy(doc_hbm.at[d], doc_v)
            for r in range(16 * NBINS // L):
                hist_v[pl.ds(r * L, L)] = zeros

            def group(g, off):
                w0 = g * L
                b = [doc_v[j, pl.ds(w0, L)] for j in range(16)]
                p = [b[4 * j] | (b[4 * j + 1] << 8) | (b[4 * j + 2] << 16)
                     | (b[4 * j + 3] << 24) for j in range(4)]
                h = jnp.zeros((L,), jnp.uint32)
                for j in range(4):
                    h = h ^ (p[j].astype(jnp.uint32) * c1 * c2)
                idxw = (h % jnp.uint32(VOCAB)).astype(jnp.int32)
                iK = idxw * (4 * C)
                iF = idxw * C
                ind = zeros
                for s in range(C):
                    t0 = plsc.load_gather(wht_v, [iK + (4 * s + 0)])
                    t1 = plsc.load_gather(wht_v, [iK + (4 * s + 1)])
                    t2 = plsc.load_gather(wht_v, [iK + (4 * s + 2)])
                    t3 = plsc.load_gather(wht_v, [iK + (4 * s + 3)])
                    fv = plsc.load_gather(wfi_v, [iF + s])
                    m = (p[0] == t0) & (p[1] == t1) & (p[2] == t2) & (p[3] == t3)
                    ind = ind + jnp.where(m, fv, 0)
                plsc.addupdate_scatter(hist_v, [laneH + ind], ones)
                keep = ind < KEEP_LT
                plsc.store_compressed(kept_v.at[pl.ds(off, L)], ind, mask=keep)
                nkeep = jnp.sum(keep, dtype=jnp.int32).astype(jnp.int32)
                return (off + nkeep).astype(jnp.int32)

            def groupN(gq, off):
                g = gq * UNROLL
                for u in range(UNROLL):
                    off = group(g + u, off)
                return off

            off = lax.fori_loop(jnp.int32(0), jnp.int32(GROUPS // UNROLL),
                                groupN, jnp.int32(0))

            npairs = jnp.maximum(off - 1, 0)
            ngrp2 = ((npairs + (2 * L - 1)) // (2 * L)).astype(jnp.int32)

            def pone(base):
                ia = iot + base
                a = plsc.load_gather(kept_v, [ia])
                b = plsc.load_gather(kept_v, [ia + 1])
                valid = ia < npairs
                bind = zeros
                for s in range(2 * C2):
                    bind = bind + jnp.where((a == bgA[s]) & (b == bgB[s]), bgF[s], 0)
                plsc.addupdate_scatter(hist_v, [laneH + bind], ones, mask=valid)

            def pgroup(i, carry):
                pone(i * (2 * L))
                pone(i * (2 * L) + L)
                return carry

            lax.fori_loop(jnp.int32(0), ngrp2, pgroup, jnp.int32(0))

            acc = [zeros] * (NBINS // L)
            for r in range(16):
                for c in range(NBINS // L):
                    acc[c] = acc[c] + hist_v[pl.ds(r * NBINS + c * L, L)]
            for c in range(NBINS // L):
                cnt_v[pl.ds(c * L, L)] = acc[c]
            pltpu.sync_copy(cnt_v, out_hbm.at[d])

    return body, KEPT_PAD


def kernel(documents, word_hash_table, word_feature_indices,
           bigram_hash_table, bigram_feature_indices):
    B, NWORDS, MWL = documents.shape
    VOCAB, C, _ = word_hash_table.shape
    _, C2, _ = bigram_hash_table.shape
    NF = 32
    KEEP_LT = NF + 1

    docs = documents.transpose(0, 2, 1)
    if docs.dtype != jnp.int32:
        docs = docs.astype(jnp.int32)
    whtf = word_hash_table.astype(jnp.int32).reshape(-1)
    wfif = word_feature_indices.astype(jnp.int32).reshape(-1)
    bhtf = bigram_hash_table.astype(jnp.int32).reshape(-1)
    bfif = bigram_feature_indices.astype(jnp.int32).reshape(-1)
    wht = jnp.zeros((_pad16(whtf.shape[0]),), jnp.int32).at[:whtf.shape[0]].set(whtf)
    wfi = jnp.zeros((_pad16(wfif.shape[0]),), jnp.int32).at[:wfif.shape[0]].set(wfif)
    bht = jnp.zeros((_pad16(bhtf.shape[0]),), jnp.int32).at[:bhtf.shape[0]].set(bhtf)
    bfi = jnp.zeros((_pad16(bfif.shape[0]),), jnp.int32).at[:bfif.shape[0]].set(bfif)

    body, KEPT_PAD = _make_body(B, NWORDS, VOCAB, C, C2, KEEP_LT)
    mesh = plsc.VectorSubcoreMesh(core_axis_name="c", subcore_axis_name="s")
    run = pl.kernel(
        body,
        mesh=mesh,
        compiler_params=pltpu.CompilerParams(use_tc_tiling_on_sc=False,
                                             needs_layout_passes=False),
        out_type=jax.ShapeDtypeStruct((B, NBINS), jnp.int32),
        scratch_types=[
            pltpu.VMEM((MWL, NWORDS), jnp.int32),
            pltpu.VMEM((wht.shape[0],), jnp.int32),
            pltpu.VMEM((wfi.shape[0],), jnp.int32),
            pltpu.VMEM((bht.shape[0],), jnp.int32),
            pltpu.VMEM((bfi.shape[0],), jnp.int32),
            pltpu.VMEM((16 * NBINS,), jnp.int32),
            pltpu.VMEM((KEPT_PAD,), jnp.int32),
            pltpu.VMEM((NBINS,), jnp.int32),
        ],
    )
    padded = run(docs, wht, wfi, bht, bfi)
    return padded[:, 1:1 + NF].astype(jnp.int64)
